# Initial kernel scaffold; baseline (speedup 1.0000x reference)
import jax
import jax.numpy as jnp
from jax.experimental import pallas as pl


def kernel(input, kernels, projection_map):
    raise NotImplementedError("write your pallas kernel here")


# speedup vs baseline: 2.2829x; 2.2829x over previous
import functools

import jax
import jax.numpy as jnp
from jax.experimental import pallas as pl
from jax.experimental.pallas import tpu as pltpu

_KH = 5
_PAD = _KH // 2
_INV_ALPHA = 10.0

_ROW0 = 8
_COL0 = 128


def _lbp_body(H, W, P, cs_ref, kys_ref, kxs_ref, x_ref, out_ref, pad_ref):
    f = pl.program_id(1)
    p = pl.program_id(2)
    idx = f * P + p
    R, L = pad_ref.shape

    first = (pl.program_id(0) == 0) & (f == 0) & (p == 0)

    @pl.when(first)
    def _():
        pad_ref[...] = jnp.zeros_like(pad_ref)

    ch = x_ref[0, 0]
    pad_ref[_ROW0:_ROW0 + H, _COL0:_COL0 + W] = ch

    ky = kys_ref[idx]
    kx = kxs_ref[idx]
    s = pad_ref[...]
    s = pltpu.roll(s, R - (_ROW0 - _PAD) - ky, 0)
    s = pltpu.roll(s, L - (_COL0 - _PAD) - kx, 1)
    nb = s[0:H, 0:W]

    bit = jax.nn.sigmoid((nb - ch) * _INV_ALPHA)
    val = jnp.exp2(p.astype(jnp.float32)) * bit

    @pl.when(p == 0)
    def _():
        out_ref[0, 0] = val

    @pl.when(p != 0)
    def _():
        out_ref[0, 0] += val


def kernel(input, kernels, projection_map):
    N, C, H, W = input.shape
    F, P = projection_map.shape

    cs = projection_map.reshape(-1).astype(jnp.int32)
    kys = kernels[..., 0].reshape(-1).astype(jnp.int32)
    kxs = kernels[..., 1].reshape(-1).astype(jnp.int32)

    body = functools.partial(_lbp_body, H, W, P)

    rows = _ROW0 + H + 8
    cols = _COL0 + H + 32

    grid_spec = pltpu.PrefetchScalarGridSpec(
        num_scalar_prefetch=3,
        grid=(N, F, P),
        in_specs=[
            pl.BlockSpec(
                (1, 1, H, W),
                lambda n, f, p, cs_r, kys_r, kxs_r: (n, cs_r[f * P + p], 0, 0),
            )
        ],
        out_specs=pl.BlockSpec(
            (1, 1, H, W),
            lambda n, f, p, cs_r, kys_r, kxs_r: (n, f, 0, 0),
        ),
        scratch_shapes=[pltpu.VMEM((rows, cols), jnp.float32)],
    )

    return pl.pallas_call(
        body,
        grid_spec=grid_spec,
        out_shape=jax.ShapeDtypeStruct((N, F, H, W), jnp.float32),
        compiler_params=pltpu.CompilerParams(
            dimension_semantics=("parallel", "arbitrary", "arbitrary"),
        ),
    )(cs, kys, kxs, input)
